# baseline (speedup 1.0000x reference)
import functools

import jax
import jax.numpy as jnp
from jax import lax
from jax.experimental import pallas as pl
from jax.experimental.pallas import tpu as pltpu
from jax.experimental.pallas import tpu_sc as plsc

BATCH = 4096
HIST = 200
EMBED = 64
MAXF = 1000000
TOTAL = BATCH * HIST

_info = plsc.get_sparse_core_info()
NC = _info.num_cores
NS = _info.num_subcores
NW = NC * NS
PER_W = TOTAL // NW

SUB = 80
K = 8
CHUNK = SUB * K
N_CHUNKS = PER_W // CHUNK

_mesh = plsc.VectorSubcoreMesh(core_axis_name="c", subcore_axis_name="s")

DEP_CHUNK = 488
DEP_N = 64
DEP_PER_W = DEP_CHUNK * DEP_N
DEP_TAIL = MAXF - DEP_PER_W * NW
DEP_TAIL_W = DEP_TAIL // 8


@functools.partial(
    pl.kernel,
    mesh=_mesh,
    out_type=jax.ShapeDtypeStruct((MAXF, 2 * EMBED), jnp.float32),
    compiler_params=pltpu.CompilerParams(use_tc_tiling_on_sc=True),
    scratch_types=[
        pltpu.VMEM((DEP_CHUNK, 2 * EMBED), jnp.float32),
        pltpu.VMEM((DEP_CHUNK, 2 * EMBED), jnp.float32),
        pltpu.SemaphoreType.DMA,
        pltpu.SemaphoreType.DMA,
        pltpu.SemaphoreType.DMA,
        pltpu.SemaphoreType.DMA,
    ],
)
def _depad(tbl_hbm, out_hbm, buf_a, buf_b, sem_ia, sem_ib, sem_oa, sem_ob):
    wid = lax.axis_index("s") * NC + lax.axis_index("c")
    row0 = wid * DEP_PER_W

    def i_fire(buf, sem, c):
        pltpu.async_copy(
            tbl_hbm.at[pl.ds(row0 + c * DEP_CHUNK, DEP_CHUNK)],
            buf.at[:, pl.ds(0, EMBED)], sem)

    def i_wait(buf, sem):
        pltpu.make_async_copy(
            tbl_hbm.at[pl.ds(row0, DEP_CHUNK)],
            buf.at[:, pl.ds(0, EMBED)], sem).wait()

    def o_fire(buf, sem, c):
        pltpu.async_copy(
            buf, out_hbm.at[pl.ds(row0 + c * DEP_CHUNK, DEP_CHUNK)], sem)

    def o_wait(buf, sem):
        pltpu.make_async_copy(
            buf, out_hbm.at[pl.ds(row0, DEP_CHUNK)], sem).wait()

    i_fire(buf_a, sem_ia, 0)
    i_wait(buf_a, sem_ia)
    i_fire(buf_b, sem_ib, 1)
    o_fire(buf_a, sem_oa, 0)

    def body(g, carry):
        o_wait(buf_a, sem_oa)
        i_fire(buf_a, sem_ia, 2 * g + 2)
        i_wait(buf_b, sem_ib)
        o_fire(buf_b, sem_ob, 2 * g + 1)
        o_wait(buf_b, sem_ob)
        i_fire(buf_b, sem_ib, 2 * g + 3)
        i_wait(buf_a, sem_ia)
        o_fire(buf_a, sem_oa, 2 * g + 2)
        return carry

    lax.fori_loop(0, DEP_N // 2 - 1, body, 0)

    o_wait(buf_a, sem_oa)
    i_wait(buf_b, sem_ib)
    o_fire(buf_b, sem_ob, DEP_N - 1)
    o_wait(buf_b, sem_ob)

    @pl.when(wid < 8)
    def _():
        t0 = NW * DEP_PER_W + wid * DEP_TAIL_W
        pltpu.sync_copy(tbl_hbm.at[pl.ds(t0, DEP_TAIL_W)],
                        buf_a.at[pl.ds(0, DEP_TAIL_W), pl.ds(0, EMBED)])
        pltpu.sync_copy(buf_a.at[pl.ds(0, DEP_TAIL_W)],
                        out_hbm.at[pl.ds(t0, DEP_TAIL_W)])


@functools.partial(
    pl.kernel,
    mesh=_mesh,
    out_type=jax.ShapeDtypeStruct((TOTAL, 2 * EMBED), jnp.float32),
    compiler_params=pltpu.CompilerParams(use_tc_tiling_on_sc=False),
    scratch_types=[
        pltpu.VMEM((PER_W,), jnp.int32),
        pltpu.VMEM((CHUNK, EMBED), jnp.float32),
        pltpu.VMEM((CHUNK, EMBED), jnp.float32),
        pltpu.SemaphoreType.DMA,
        pltpu.SemaphoreType.DMA,
        pltpu.SemaphoreType.DMA,
        pltpu.SemaphoreType.DMA,
    ],
)
def _gather_all(idx_hbm, table_hbm, out_hbm, idx_v, rows_a, rows_b,
                sem_ga, sem_gb, sem_oa, sem_ob):
    wid = lax.axis_index("s") * NC + lax.axis_index("c")
    out_row0 = wid * PER_W

    def g_fire(rows, sem, c):
        for j in range(K):
            pltpu.async_copy(
                table_hbm.at[idx_v.at[pl.ds(c * CHUNK + j * SUB, SUB)]],
                rows.at[pl.ds(j * SUB, SUB)],
                sem,
            )

    def g_wait(rows, sem):
        for j in range(K):
            pltpu.make_async_copy(
                table_hbm.at[idx_v.at[pl.ds(0, SUB)]],
                rows.at[pl.ds(j * SUB, SUB)],
                sem,
            ).wait()

    def o_fire(rows, sem, c):
        pltpu.async_copy(
            rows,
            out_hbm.at[pl.ds(out_row0 + c * CHUNK, CHUNK), pl.ds(0, EMBED)],
            sem)

    def o_wait(rows, sem):
        pltpu.make_async_copy(
            rows,
            out_hbm.at[pl.ds(out_row0, CHUNK), pl.ds(0, EMBED)],
            sem).wait()

    pltpu.sync_copy(idx_hbm.at[pl.ds(wid * PER_W, PER_W)], idx_v)

    g_fire(rows_a, sem_ga, 0)
    g_wait(rows_a, sem_ga)
    g_fire(rows_b, sem_gb, 1)
    o_fire(rows_a, sem_oa, 0)

    def body(g, carry):
        o_wait(rows_a, sem_oa)
        g_fire(rows_a, sem_ga, 2 * g + 2)
        g_wait(rows_b, sem_gb)
        o_fire(rows_b, sem_ob, 2 * g + 1)
        o_wait(rows_b, sem_ob)
        g_fire(rows_b, sem_gb, 2 * g + 3)
        g_wait(rows_a, sem_ga)
        o_fire(rows_a, sem_oa, 2 * g + 2)
        return carry

    lax.fori_loop(0, N_CHUNKS // 2 - 1, body, 0)

    o_wait(rows_a, sem_oa)
    g_wait(rows_b, sem_gb)
    o_fire(rows_b, sem_ob, N_CHUNKS - 1)
    o_wait(rows_b, sem_ob)


def kernel(x, table):
    j = jnp.arange(2 * EMBED)
    e = jnp.arange(EMBED)
    w = jnp.arange(2)
    sel = ((e[None, :, None] == j[:, None, None] % EMBED)
           & (w[None, None, :] == j[:, None, None] // EMBED)
           ).astype(jnp.float32)
    conv = lax.conv_general_dilated(
        table.T[None],
        sel, window_strides=(2,), padding="VALID",
        dimension_numbers=("NCH", "OIH", "NCH"),
        precision=jax.lax.Precision.HIGHEST)
    table2 = conv[0].T.reshape(MAXF, EMBED)
    idx = x.reshape(TOTAL).astype(jnp.int32)
    out = _gather_all(idx, table2)
    return out.reshape(BATCH, HIST, 2 * EMBED)[:, :, :EMBED]


# speedup vs baseline: 1.9521x; 1.9521x over previous
import functools

import jax
import jax.numpy as jnp
from jax import lax
from jax.experimental import pallas as pl
from jax.experimental.pallas import tpu as pltpu
from jax.experimental.pallas import tpu_sc as plsc

BATCH = 4096
HIST = 200
EMBED = 64
MAXF = 1000000
TOTAL = BATCH * HIST

_info = plsc.get_sparse_core_info()
NC = _info.num_cores
NS = _info.num_subcores
NW = NC * NS
PER_W = TOTAL // NW

SUB = 80
K = 8
CHUNK = SUB * K
N_CHUNKS = PER_W // CHUNK

_mesh = plsc.VectorSubcoreMesh(core_axis_name="c", subcore_axis_name="s")

DEP_CHUNK = 488
DEP_N = 64
DEP_PER_W = DEP_CHUNK * DEP_N
DEP_TAIL = MAXF - DEP_PER_W * NW
DEP_TAIL_W = DEP_TAIL // 8


@functools.partial(
    pl.kernel,
    mesh=_mesh,
    out_type=jax.ShapeDtypeStruct((MAXF, 2 * EMBED), jnp.float32),
    compiler_params=pltpu.CompilerParams(use_tc_tiling_on_sc=True),
    scratch_types=[
        pltpu.VMEM((DEP_CHUNK, 2 * EMBED), jnp.float32),
        pltpu.VMEM((DEP_CHUNK, 2 * EMBED), jnp.float32),
        pltpu.SemaphoreType.DMA,
        pltpu.SemaphoreType.DMA,
        pltpu.SemaphoreType.DMA,
        pltpu.SemaphoreType.DMA,
    ],
)
def _depad(tbl_hbm, out_hbm, buf_a, buf_b, sem_ia, sem_ib, sem_oa, sem_ob):
    wid = lax.axis_index("s") * NC + lax.axis_index("c")
    row0 = wid * DEP_PER_W

    def i_fire(buf, sem, c):
        pltpu.async_copy(
            tbl_hbm.at[pl.ds(row0 + c * DEP_CHUNK, DEP_CHUNK)],
            buf.at[:, pl.ds(0, EMBED)], sem)

    def i_wait(buf, sem):
        pltpu.make_async_copy(
            tbl_hbm.at[pl.ds(row0, DEP_CHUNK)],
            buf.at[:, pl.ds(0, EMBED)], sem).wait()

    def o_fire(buf, sem, c):
        pltpu.async_copy(
            buf, out_hbm.at[pl.ds(row0 + c * DEP_CHUNK, DEP_CHUNK)], sem)

    def o_wait(buf, sem):
        pltpu.make_async_copy(
            buf, out_hbm.at[pl.ds(row0, DEP_CHUNK)], sem).wait()

    i_fire(buf_a, sem_ia, 0)
    i_wait(buf_a, sem_ia)
    i_fire(buf_b, sem_ib, 1)
    o_fire(buf_a, sem_oa, 0)

    def body(g, carry):
        o_wait(buf_a, sem_oa)
        i_fire(buf_a, sem_ia, 2 * g + 2)
        i_wait(buf_b, sem_ib)
        o_fire(buf_b, sem_ob, 2 * g + 1)
        o_wait(buf_b, sem_ob)
        i_fire(buf_b, sem_ib, 2 * g + 3)
        i_wait(buf_a, sem_ia)
        o_fire(buf_a, sem_oa, 2 * g + 2)
        return carry

    lax.fori_loop(0, DEP_N // 2 - 1, body, 0)

    o_wait(buf_a, sem_oa)
    i_wait(buf_b, sem_ib)
    o_fire(buf_b, sem_ob, DEP_N - 1)
    o_wait(buf_b, sem_ob)

    @pl.when(wid < 8)
    def _():
        t0 = NW * DEP_PER_W + wid * DEP_TAIL_W
        pltpu.sync_copy(tbl_hbm.at[pl.ds(t0, DEP_TAIL_W)],
                        buf_a.at[pl.ds(0, DEP_TAIL_W), pl.ds(0, EMBED)])
        pltpu.sync_copy(buf_a.at[pl.ds(0, DEP_TAIL_W)],
                        out_hbm.at[pl.ds(t0, DEP_TAIL_W)])


@functools.partial(
    pl.kernel,
    mesh=_mesh,
    out_type=jax.ShapeDtypeStruct((TOTAL, 2 * EMBED), jnp.float32),
    compiler_params=pltpu.CompilerParams(use_tc_tiling_on_sc=False),
    scratch_types=[
        pltpu.VMEM((PER_W,), jnp.int32),
        pltpu.VMEM((CHUNK, EMBED), jnp.float32),
        pltpu.VMEM((CHUNK, EMBED), jnp.float32),
        pltpu.SemaphoreType.DMA,
        pltpu.SemaphoreType.DMA,
        pltpu.SemaphoreType.DMA,
        pltpu.SemaphoreType.DMA,
    ],
)
def _gather_all(idx_hbm, table_hbm, out_hbm, idx_v, rows_a, rows_b,
                sem_ga, sem_gb, sem_oa, sem_ob):
    wid = lax.axis_index("s") * NC + lax.axis_index("c")
    out_row0 = wid * PER_W

    def g_fire(rows, sem, c):
        for j in range(K):
            pltpu.async_copy(
                table_hbm.at[idx_v.at[pl.ds(c * CHUNK + j * SUB, SUB)]],
                rows.at[pl.ds(j * SUB, SUB)],
                sem,
            )

    def g_wait(rows, sem):
        for j in range(K):
            pltpu.make_async_copy(
                table_hbm.at[idx_v.at[pl.ds(0, SUB)]],
                rows.at[pl.ds(j * SUB, SUB)],
                sem,
            ).wait()

    def o_fire(rows, sem, c):
        pltpu.async_copy(
            rows,
            out_hbm.at[pl.ds(out_row0 + c * CHUNK, CHUNK), pl.ds(0, EMBED)],
            sem)

    def o_wait(rows, sem):
        pltpu.make_async_copy(
            rows,
            out_hbm.at[pl.ds(out_row0, CHUNK), pl.ds(0, EMBED)],
            sem).wait()

    pltpu.sync_copy(idx_hbm.at[pl.ds(wid * PER_W, PER_W)], idx_v)

    g_fire(rows_a, sem_ga, 0)
    g_wait(rows_a, sem_ga)
    g_fire(rows_b, sem_gb, 1)
    o_fire(rows_a, sem_oa, 0)

    def body(g, carry):
        o_wait(rows_a, sem_oa)
        g_fire(rows_a, sem_ga, 2 * g + 2)
        g_wait(rows_b, sem_gb)
        o_fire(rows_b, sem_ob, 2 * g + 1)
        o_wait(rows_b, sem_ob)
        g_fire(rows_b, sem_gb, 2 * g + 3)
        g_wait(rows_a, sem_ga)
        o_fire(rows_a, sem_oa, 2 * g + 2)
        return carry

    lax.fori_loop(0, N_CHUNKS // 2 - 1, body, 0)

    o_wait(rows_a, sem_oa)
    g_wait(rows_b, sem_gb)
    o_fire(rows_b, sem_ob, N_CHUNKS - 1)
    o_wait(rows_b, sem_ob)


def kernel(x, table):
    pad_mat = jnp.eye(EMBED, 2 * EMBED, dtype=jnp.float32)
    table2 = jnp.matmul(
        table, pad_mat, precision=jax.lax.Precision.HIGHEST
    ).reshape(2 * MAXF, EMBED)
    idx = x.reshape(TOTAL).astype(jnp.int32) * 2
    out = _gather_all(idx, table2)
    return out.reshape(BATCH, HIST, 2 * EMBED)[:, :, :EMBED]


# speedup vs baseline: 1.9552x; 1.0016x over previous
import functools

import jax
import jax.numpy as jnp
from jax import lax
from jax.experimental import pallas as pl
from jax.experimental.pallas import tpu as pltpu
from jax.experimental.pallas import tpu_sc as plsc

BATCH = 4096
HIST = 200
EMBED = 64
MAXF = 1000000
TOTAL = BATCH * HIST

_info = plsc.get_sparse_core_info()
NC = _info.num_cores
NS = _info.num_subcores
NW = NC * NS
PER_W = TOTAL // NW

SUB = 80
K = 8
CHUNK = SUB * K
N_CHUNKS = PER_W // CHUNK

_mesh = plsc.VectorSubcoreMesh(core_axis_name="c", subcore_axis_name="s")


@functools.partial(
    pl.kernel,
    mesh=_mesh,
    out_type=jax.ShapeDtypeStruct((TOTAL, 2 * EMBED), jnp.float32),
    compiler_params=pltpu.CompilerParams(use_tc_tiling_on_sc=False),
    scratch_types=[
        pltpu.VMEM((PER_W,), jnp.int32),
        pltpu.VMEM((CHUNK, EMBED), jnp.float32),
        pltpu.VMEM((CHUNK, EMBED), jnp.float32),
        pltpu.SemaphoreType.DMA,
        pltpu.SemaphoreType.DMA,
        pltpu.SemaphoreType.DMA,
        pltpu.SemaphoreType.DMA,
    ],
)
def _gather_all(idx_hbm, table_hbm, out_hbm, idx_v, rows_a, rows_b,
                sem_ga, sem_gb, sem_oa, sem_ob):
    wid = lax.axis_index("s") * NC + lax.axis_index("c")
    out_row0 = wid * PER_W

    def g_fire(rows, sem, c):
        for j in range(K):
            pltpu.async_copy(
                table_hbm.at[idx_v.at[pl.ds(c * CHUNK + j * SUB, SUB)]],
                rows.at[pl.ds(j * SUB, SUB)],
                sem,
            )

    def g_wait(rows, sem):
        for j in range(K):
            pltpu.make_async_copy(
                table_hbm.at[idx_v.at[pl.ds(0, SUB)]],
                rows.at[pl.ds(j * SUB, SUB)],
                sem,
            ).wait()

    def o_fire(rows, sem, c):
        pltpu.async_copy(
            rows,
            out_hbm.at[pl.ds(out_row0 + c * CHUNK, CHUNK), pl.ds(0, EMBED)],
            sem)

    def o_wait(rows, sem):
        pltpu.make_async_copy(
            rows,
            out_hbm.at[pl.ds(out_row0, CHUNK), pl.ds(0, EMBED)],
            sem).wait()

    pltpu.sync_copy(idx_hbm.at[pl.ds(wid * PER_W, PER_W)], idx_v)

    g_fire(rows_a, sem_ga, 0)
    g_wait(rows_a, sem_ga)
    g_fire(rows_b, sem_gb, 1)
    o_fire(rows_a, sem_oa, 0)

    def body(g, carry):
        o_wait(rows_a, sem_oa)
        g_fire(rows_a, sem_ga, 2 * g + 2)
        g_wait(rows_b, sem_gb)
        o_fire(rows_b, sem_ob, 2 * g + 1)
        o_wait(rows_b, sem_ob)
        g_fire(rows_b, sem_gb, 2 * g + 3)
        g_wait(rows_a, sem_ga)
        o_fire(rows_a, sem_oa, 2 * g + 2)
        return carry

    lax.fori_loop(0, N_CHUNKS // 2 - 1, body, 0)

    o_wait(rows_a, sem_oa)
    g_wait(rows_b, sem_gb)
    o_fire(rows_b, sem_ob, N_CHUNKS - 1)
    o_wait(rows_b, sem_ob)


def kernel(x, table):
    pad_mat = jnp.eye(EMBED, 2 * EMBED, dtype=jnp.float32)
    table2 = jnp.matmul(
        table, pad_mat, precision=jax.lax.Precision.HIGHEST
    ).reshape(2 * MAXF, EMBED)
    idx = x.reshape(TOTAL).astype(jnp.int32) * 2
    out = _gather_all(idx, table2)
    return out.reshape(BATCH, HIST, 2 * EMBED)[:, :, :EMBED]
